# Initial kernel scaffold; baseline (speedup 1.0000x reference)
import functools

import jax
import jax.numpy as jnp
from jax import lax
from jax.experimental import pallas as pl
from jax.experimental.pallas import tpu as pltpu
from jax.experimental.pallas import tpu_sc as plsc

NUM_ACT = 2048
SIZE = 51
RANGE_ = 4.0
GRID = 2.0 * RANGE_ / (SIZE - 1)
HALF = SIZE // 2
TABLE = NUM_ACT * SIZE

NC, NS, L = 2, 16, 16
NW = NC * NS
ROWS_PER_CHUNK = 2
CHUNK = ROWS_PER_CHUNK * NUM_ACT


def _project_body(cs_ref, out_ref):
    cs = cs_ref[...]
    slopes = jnp.clip(cs[:, 1:] - cs[:, :-1], 0.0, jnp.float32(GRID))
    k = lax.broadcasted_iota(jnp.int32, (SIZE - 1, SIZE), 0)
    j = lax.broadcasted_iota(jnp.int32, (SIZE - 1, SIZE), 1)
    m = (j > k).astype(jnp.float32)
    cum = lax.dot_general(
        slopes, m, (((1,), (0,)), ((), ())),
        preferred_element_type=jnp.float32,
        precision=lax.Precision.HIGHEST)
    out_ref[...] = cum - cum[:, HALF:HALF + 1]


def _make_sc_kernel(n_elems):
    chunks_total = n_elems // CHUNK
    ch_per_w = chunks_total // NW
    mesh = plsc.VectorSubcoreMesh(
        core_axis_name="c", subcore_axis_name="s",
        num_cores=NC, num_subcores=NS)

    @functools.partial(
        pl.kernel,
        out_type=jax.ShapeDtypeStruct((n_elems,), jnp.float32),
        mesh=mesh,
        scratch_types=[
            pltpu.VMEM((TABLE,), jnp.float32),
            pltpu.VMEM((NUM_ACT,), jnp.float32),
            pltpu.VMEM((NUM_ACT,), jnp.float32),
            pltpu.VMEM((CHUNK,), jnp.float32),
            pltpu.VMEM((CHUNK,), jnp.float32),
            pltpu.VMEM((CHUNK,), jnp.float32),
            pltpu.VMEM((CHUNK,), jnp.float32),
            pltpu.SemaphoreType.DMA,
            pltpu.SemaphoreType.DMA,
            pltpu.SemaphoreType.DMA,
            pltpu.SemaphoreType.DMA,
        ],
    )
    def sc_kernel(x_hbm, cv_hbm, pm_hbm, inv_hbm, out_hbm,
                  table, pm_v, inv_v, xb0, xb1, ob0, ob1,
                  sin0, sin1, sout0, sout1):
        wid = lax.axis_index("s") * NC + lax.axis_index("c")
        pltpu.sync_copy(cv_hbm, table)
        pltpu.sync_copy(pm_hbm, pm_v)
        pltpu.sync_copy(inv_hbm, inv_v)
        base = wid * (ch_per_w * CHUNK)
        xbs = (xb0, xb1)
        obs = (ob0, ob1)
        sins = (sin0, sin1)
        souts = (sout0, sout1)

        def start_in(c, b):
            pltpu.async_copy(
                x_hbm.at[pl.ds(base + c * CHUNK, CHUNK)], xbs[b], sins[b])

        def start_out(c, b):
            pltpu.async_copy(
                obs[b], out_hbm.at[pl.ds(base + c * CHUNK, CHUNK)], souts[b])

        def wait_in(b):
            pltpu.make_async_copy(
                x_hbm.at[pl.ds(base, CHUNK)], xbs[b], sins[b]).wait()

        def wait_out(b):
            pltpu.make_async_copy(
                obs[b], out_hbm.at[pl.ds(base, CHUNK)], souts[b]).wait()

        iota51 = lax.iota(jnp.int32, (L,)) * SIZE

        def compute(b):
            xb = xbs[b]
            ob = obs[b]

            @pl.loop(0, NUM_ACT // L)
            def _(i):
                col = i * L
                pm = pm_v[pl.ds(col, L)]
                iv = inv_v[pl.ds(col, L)]
                chv = iota51 + col * SIZE
                for r in range(ROWS_PER_CHUNK):
                    o = r * NUM_ACT + col
                    xv = xb[pl.ds(o, L)]
                    u = xv * pm + jnp.float32(HALF)
                    uc = jnp.minimum(jnp.maximum(u, 0.0),
                                     jnp.float32(SIZE - 2))
                    fl = uc.astype(jnp.int32)
                    frac = u - fl.astype(jnp.float32)
                    idx = chv + fl
                    a = plsc.load_gather(table, [idx])
                    c2 = plsc.load_gather(table, [idx + 1])
                    ob[pl.ds(o, L)] = (a + (c2 - a) * frac) * iv

        start_in(0, 0)
        start_in(1, 1)

        @pl.loop(0, ch_per_w, step=2)
        def _(c):
            for b in range(2):
                cc = c + b
                wait_in(b)

                @pl.when(cc >= 2)
                def _():
                    wait_out(b)

                compute(b)
                start_out(cc, b)

                @pl.when(cc + 2 < ch_per_w)
                def _():
                    start_in(cc + 2, b)

        wait_out(0)
        wait_out(1)

    return sc_kernel


def kernel(input, coefficients_vect, scaling_coeffs_vect):
    b, c = input.shape
    cs = coefficients_vect.reshape(NUM_ACT, SIZE)
    cv = pl.pallas_call(
        _project_body,
        out_shape=jax.ShapeDtypeStruct((NUM_ACT, SIZE), jnp.float32),
    )(cs).reshape(-1)
    s = scaling_coeffs_vect.reshape(NUM_ACT)
    pm = s * jnp.float32(1.0 / GRID)
    inv = 1.0 / s
    out = _make_sc_kernel(b * c)(input.reshape(-1), cv, pm, inv)
    return out.reshape(b, c)


# speedup vs baseline: 379.6399x; 379.6399x over previous
import functools

import jax
import jax.numpy as jnp
from jax import lax
from jax.experimental import pallas as pl
from jax.experimental.pallas import tpu as pltpu
from jax.experimental.pallas import tpu_sc as plsc

NUM_ACT = 2048
SIZE = 51
RANGE_ = 4.0
GRID = 2.0 * RANGE_ / (SIZE - 1)
HALF = SIZE // 2
TABLE = NUM_ACT * SIZE

NC, NS, L = 2, 16, 16
NW = NC * NS
ROWS_PER_CHUNK = 2
CHUNK = ROWS_PER_CHUNK * NUM_ACT


def _project_body(cs_ref, out_ref):
    cs = cs_ref[...]
    slopes = jnp.clip(cs[:, 1:] - cs[:, :-1], 0.0, jnp.float32(GRID))
    k = lax.broadcasted_iota(jnp.int32, (SIZE - 1, SIZE), 0)
    j = lax.broadcasted_iota(jnp.int32, (SIZE - 1, SIZE), 1)
    m = (j > k).astype(jnp.float32)
    cum = lax.dot_general(
        slopes, m, (((1,), (0,)), ((), ())),
        preferred_element_type=jnp.float32,
        precision=lax.Precision.HIGHEST)
    out_ref[...] = cum - cum[:, HALF:HALF + 1]


def _make_sc_kernel(n_elems):
    chunks_total = n_elems // CHUNK
    ch_per_w = chunks_total // NW
    mesh = plsc.VectorSubcoreMesh(
        core_axis_name="c", subcore_axis_name="s",
        num_cores=NC, num_subcores=NS)

    @functools.partial(
        pl.kernel,
        out_type=jax.ShapeDtypeStruct((n_elems,), jnp.float32),
        mesh=mesh,
        compiler_params=pltpu.CompilerParams(needs_layout_passes=False),
        scratch_types=[
            pltpu.VMEM((TABLE,), jnp.float32),
            pltpu.VMEM((NUM_ACT,), jnp.float32),
            pltpu.VMEM((NUM_ACT,), jnp.float32),
            pltpu.VMEM((CHUNK,), jnp.float32),
            pltpu.VMEM((CHUNK,), jnp.float32),
            pltpu.VMEM((CHUNK,), jnp.float32),
            pltpu.VMEM((CHUNK,), jnp.float32),
            pltpu.SemaphoreType.DMA,
            pltpu.SemaphoreType.DMA,
            pltpu.SemaphoreType.DMA,
            pltpu.SemaphoreType.DMA,
        ],
    )
    def sc_kernel(x_hbm, cv_hbm, pm_hbm, inv_hbm, out_hbm,
                  table, pm_v, inv_v, xb0, xb1, ob0, ob1,
                  sin0, sin1, sout0, sout1):
        wid = lax.axis_index("s") * NC + lax.axis_index("c")
        pltpu.sync_copy(cv_hbm, table)
        pltpu.sync_copy(pm_hbm, pm_v)
        pltpu.sync_copy(inv_hbm, inv_v)
        base = wid * (ch_per_w * CHUNK)
        xbs = (xb0, xb1)
        obs = (ob0, ob1)
        sins = (sin0, sin1)
        souts = (sout0, sout1)

        def start_in(c, b):
            pltpu.async_copy(
                x_hbm.at[pl.ds(base + c * CHUNK, CHUNK)], xbs[b], sins[b])

        def start_out(c, b):
            pltpu.async_copy(
                obs[b], out_hbm.at[pl.ds(base + c * CHUNK, CHUNK)], souts[b])

        def wait_in(b):
            pltpu.make_async_copy(
                x_hbm.at[pl.ds(base, CHUNK)], xbs[b], sins[b]).wait()

        def wait_out(b):
            pltpu.make_async_copy(
                obs[b], out_hbm.at[pl.ds(base, CHUNK)], souts[b]).wait()

        iota51 = lax.iota(jnp.int32, L) * SIZE

        def compute(b):
            xb = xbs[b]
            ob = obs[b]

            @pl.loop(0, NUM_ACT // L)
            def _(i):
                col = i * L
                pm = pm_v[pl.ds(col, L)]
                iv = inv_v[pl.ds(col, L)]
                chv = iota51 + col * SIZE
                for r in range(ROWS_PER_CHUNK):
                    o = r * NUM_ACT + col
                    xv = xb[pl.ds(o, L)]
                    u = xv * pm + jnp.float32(HALF)
                    uc = jnp.minimum(jnp.maximum(u, 0.0),
                                     jnp.float32(SIZE - 2))
                    fl = uc.astype(jnp.int32)
                    frac = u - fl.astype(jnp.float32)
                    idx = chv + fl
                    a = plsc.load_gather(table, [idx])
                    c2 = plsc.load_gather(table, [idx + 1])
                    ob[pl.ds(o, L)] = (a + (c2 - a) * frac) * iv

        start_in(0, 0)
        start_in(1, 1)

        @pl.loop(0, ch_per_w, step=2)
        def _(c):
            for b in range(2):
                cc = c + b
                wait_in(b)

                @pl.when(cc >= 2)
                def _():
                    wait_out(b)

                compute(b)
                start_out(cc, b)

                @pl.when(cc + 2 < ch_per_w)
                def _():
                    start_in(cc + 2, b)

        wait_out(0)
        wait_out(1)

    return sc_kernel


def kernel(input, coefficients_vect, scaling_coeffs_vect):
    b, c = input.shape
    cs = coefficients_vect.reshape(NUM_ACT, SIZE)
    cv = pl.pallas_call(
        _project_body,
        out_shape=jax.ShapeDtypeStruct((NUM_ACT, SIZE), jnp.float32),
    )(cs).reshape(-1)
    s = scaling_coeffs_vect.reshape(NUM_ACT)
    pm = s * jnp.float32(1.0 / GRID)
    inv = 1.0 / s
    out = _make_sc_kernel(b * c)(input.reshape(-1), cv, pm, inv)
    return out.reshape(b, c)


# speedup vs baseline: 1195.3645x; 3.1487x over previous
import functools

import jax
import jax.numpy as jnp
from jax import lax
from jax.experimental import pallas as pl
from jax.experimental.pallas import tpu as pltpu
from jax.experimental.pallas import tpu_sc as plsc

NUM_ACT = 2048
SIZE = 51
RANGE_ = 4.0
GRID = 2.0 * RANGE_ / (SIZE - 1)
HALF = SIZE // 2
TABLE = NUM_ACT * SIZE

NC, NS, L = 2, 16, 16
NW = NC * NS
ROWS_PER_CHUNK = 2
CHUNK = ROWS_PER_CHUNK * NUM_ACT


def _project_body(cs_ref, out_ref):
    cs = cs_ref[...]
    slopes = jnp.clip(cs[:, 1:] - cs[:, :-1], 0.0, jnp.float32(GRID))
    k = lax.broadcasted_iota(jnp.int32, (SIZE - 1, SIZE), 0)
    j = lax.broadcasted_iota(jnp.int32, (SIZE - 1, SIZE), 1)
    m = (j > k).astype(jnp.float32)
    cum = lax.dot_general(
        slopes, m, (((1,), (0,)), ((), ())),
        preferred_element_type=jnp.float32,
        precision=lax.Precision.HIGHEST)
    out_ref[...] = cum - cum[:, HALF:HALF + 1]


def _make_sc_kernel(n_elems):
    chunks_total = n_elems // CHUNK
    ch_per_w = chunks_total // NW
    mesh = plsc.VectorSubcoreMesh(
        core_axis_name="c", subcore_axis_name="s",
        num_cores=NC, num_subcores=NS)

    @functools.partial(
        pl.kernel,
        out_type=jax.ShapeDtypeStruct((n_elems,), jnp.float32),
        mesh=mesh,
        compiler_params=pltpu.CompilerParams(needs_layout_passes=False),
        scratch_types=[
            pltpu.VMEM((TABLE,), jnp.float32),
            pltpu.VMEM((NUM_ACT,), jnp.float32),
            pltpu.VMEM((NUM_ACT,), jnp.float32),
            pltpu.VMEM((CHUNK,), jnp.float32),
            pltpu.VMEM((CHUNK,), jnp.float32),
            pltpu.VMEM((CHUNK,), jnp.float32),
            pltpu.VMEM((CHUNK,), jnp.float32),
            pltpu.SemaphoreType.DMA,
            pltpu.SemaphoreType.DMA,
            pltpu.SemaphoreType.DMA,
            pltpu.SemaphoreType.DMA,
        ],
    )
    def sc_kernel(x_hbm, cv_hbm, pm_hbm, inv_hbm, out_hbm,
                  table, pm_v, inv_v, xb0, xb1, ob0, ob1,
                  sin0, sin1, sout0, sout1):
        wid = lax.axis_index("s") * NC + lax.axis_index("c")
        pltpu.sync_copy(cv_hbm, table)
        pltpu.sync_copy(pm_hbm, pm_v)
        pltpu.sync_copy(inv_hbm, inv_v)
        base = wid * (ch_per_w * CHUNK)
        xbs = (xb0, xb1)
        obs = (ob0, ob1)
        sins = (sin0, sin1)
        souts = (sout0, sout1)

        def start_in(c, b):
            pltpu.async_copy(
                x_hbm.at[pl.ds(base + c * CHUNK, CHUNK)], xbs[b], sins[b])

        def start_out(c, b):
            pltpu.async_copy(
                obs[b], out_hbm.at[pl.ds(base + c * CHUNK, CHUNK)], souts[b])

        def wait_in(b):
            pltpu.make_async_copy(
                x_hbm.at[pl.ds(base, CHUNK)], xbs[b], sins[b]).wait()

        def wait_out(b):
            pltpu.make_async_copy(
                obs[b], out_hbm.at[pl.ds(base, CHUNK)], souts[b]).wait()

        iota51 = lax.iota(jnp.int32, L) * SIZE

        def compute(b):
            xb = xbs[b]
            ob = obs[b]

            @plsc.parallel_loop(0, NUM_ACT // L, unroll=4)
            def _(i):
                col = i * L
                pm = pm_v[pl.ds(col, L)]
                iv = inv_v[pl.ds(col, L)]
                chv = iota51 + col * SIZE
                for r in range(ROWS_PER_CHUNK):
                    o = r * NUM_ACT + col
                    xv = xb[pl.ds(o, L)]
                    u = xv * pm + jnp.float32(HALF)
                    uc = jnp.minimum(jnp.maximum(u, 0.0),
                                     jnp.float32(SIZE - 2))
                    fl = uc.astype(jnp.int32)
                    frac = u - fl.astype(jnp.float32)
                    idx = chv + fl
                    a = plsc.load_gather(table, [idx])
                    c2 = plsc.load_gather(table, [idx + 1])
                    ob[pl.ds(o, L)] = (a + (c2 - a) * frac) * iv

        start_in(0, 0)
        start_in(1, 1)

        @pl.loop(0, ch_per_w, step=2)
        def _(c):
            for b in range(2):
                cc = c + b
                wait_in(b)

                @pl.when(cc >= 2)
                def _():
                    wait_out(b)

                compute(b)
                start_out(cc, b)

                @pl.when(cc + 2 < ch_per_w)
                def _():
                    start_in(cc + 2, b)

        wait_out(0)
        wait_out(1)

    return sc_kernel


def kernel(input, coefficients_vect, scaling_coeffs_vect):
    b, c = input.shape
    cs = coefficients_vect.reshape(NUM_ACT, SIZE)
    cv = pl.pallas_call(
        _project_body,
        out_shape=jax.ShapeDtypeStruct((NUM_ACT, SIZE), jnp.float32),
    )(cs).reshape(-1)
    s = scaling_coeffs_vect.reshape(NUM_ACT)
    pm = s * jnp.float32(1.0 / GRID)
    inv = 1.0 / s
    out = _make_sc_kernel(b * c)(input.reshape(-1), cv, pm, inv)
    return out.reshape(b, c)
